# baseline (speedup 1.0000x reference)
import functools

import jax
import jax.numpy as jnp
from jax import lax
from jax.experimental import pallas as pl
from jax.experimental.pallas import tpu as pltpu
from jax.experimental.pallas import tpu_sc as plsc

_B, _F, _K, _D = 16384, 100, 50, 64
_NW = 32
_LANES = _B // _NW
_HALF = _D // 2
_NCHUNK = 2 * _K
_NBUF = 4


def _make_sc_kernel():
    mesh = plsc.VectorSubcoreMesh(core_axis_name="c", subcore_axis_name="s")

    @functools.partial(
        pl.kernel,
        mesh=mesh,
        out_type=jax.ShapeDtypeStruct((_K, _D, _B), jnp.float32),
        scratch_types=[
            pltpu.VMEM((_NBUF, _HALF, _LANES), jnp.float32),
            [pltpu.SemaphoreType.DMA] * _NBUF,
            [pltpu.SemaphoreType.DMA] * _NBUF,
        ],
    )
    def sc_copy(in_hbm, out_hbm, buf, rsem, wsem):
        wid = lax.axis_index("s") * 2 + lax.axis_index("c")
        lane0 = wid * _LANES

        def start_read(slab, h, slot):
            pltpu.async_copy(
                in_hbm.at[2 * slab, pl.ds(h * _HALF, _HALF),
                          pl.ds(lane0, _LANES)],
                buf.at[slot], rsem[slot])

        def start_write(slab, h, slot):
            pltpu.async_copy(
                buf.at[slot],
                out_hbm.at[slab, pl.ds(h * _HALF, _HALF),
                           pl.ds(lane0, _LANES)],
                wsem[slot])

        def wait_read(slot):
            pltpu.make_async_copy(
                in_hbm.at[0, pl.ds(0, _HALF), pl.ds(lane0, _LANES)],
                buf.at[slot], rsem[slot]).wait()

        def wait_write(slot):
            pltpu.make_async_copy(
                buf.at[slot],
                out_hbm.at[0, pl.ds(0, _HALF), pl.ds(lane0, _LANES)],
                wsem[slot]).wait()

        start_read(0, 0, 0)
        start_read(0, 1, 1)
        for k in (0, 1):
            wait_read(k)
            start_write(0, k, k)
            start_read((k + 2) // 2, k, (k + 2) % _NBUF)

        @pl.loop(2, _NCHUNK - 2, step=_NBUF)
        def _(k0):
            s0 = k0 // 2
            for b in range(_NBUF):
                slot = (2 + b) % _NBUF
                nslot = (2 + b + 2) % _NBUF
                slab = s0 + (b // 2)
                h = b % 2
                wait_read(slot)
                wait_write(nslot)
                start_write(slab, h, slot)
                start_read(slab + 1, h, nslot)

        for k in (_NCHUNK - 2, _NCHUNK - 1):
            slot = k % _NBUF
            wait_read(slot)
            wait_write((k + 2) % _NBUF)
            start_write(k // 2, k % 2, slot)
        for k in (_NCHUNK - 2, _NCHUNK - 1):
            wait_write(k % _NBUF)

    return sc_copy


_SC_KERNEL = _make_sc_kernel()


def kernel(tensor):
    x_t = jnp.transpose(tensor, (1, 2, 0))
    out_t = _SC_KERNEL(x_t)
    return jnp.transpose(out_t, (2, 0, 1))
